# baseline (speedup 1.0000x reference)
import functools

import jax
import jax.numpy as jnp
from jax import lax
from jax.experimental import pallas as pl
from jax.experimental.pallas import tpu as pltpu
from jax.experimental.pallas import tpu_sc as plsc

N = 10000
E = 160000
D_IN = 256
D_H = 256
D_OUT = 128

NC = 2
NS = 16
CHUNK = 128
CH = 40

NP = 10240
ROWS_PER_TILE = NP // NS
DUMMY = N

EPAD = NC * NS * CH * CHUNK


@functools.cache
def _mesh():
    return plsc.VectorSubcoreMesh(
        core_axis_name="c", subcore_axis_name="s",
        num_cores=NC, num_subcores=NS)


@functools.cache
def _make_deg():
    DW = 16

    @functools.partial(
        pl.kernel,
        out_type=jax.ShapeDtypeStruct((NC, NP, DW), jnp.float32),
        mesh=_mesh(),
        scratch_types=[
            pltpu.VMEM((CH, CHUNK), jnp.int32),
            pltpu.VMEM((CHUNK, DW), jnp.float32),
            pltpu.VMEM_SHARED((NP, DW), jnp.float32),
        ],
    )
    def deg_kernel(didx_hbm, out_hbm, dix_v, ones_v, acc_s):
        c = lax.axis_index("c")
        s = lax.axis_index("s")
        pltpu.sync_copy(didx_hbm.at[c, s], dix_v)

        @pl.loop(0, CHUNK)
        def _(r):
            ones_v[r, :] = jnp.zeros((DW,), jnp.float32)

        row0 = s * ROWS_PER_TILE
        for i in range(ROWS_PER_TILE // CHUNK):
            pltpu.sync_copy(ones_v, acc_s.at[pl.ds(row0 + i * CHUNK, CHUNK)])

        @pl.loop(0, CHUNK)
        def _(r):
            ones_v[r, :] = jnp.ones((DW,), jnp.float32)

        plsc.subcore_barrier()

        @pl.loop(0, CH)
        def _(j):
            pltpu.sync_copy(ones_v, acc_s.at[dix_v.at[j]], add=True)

        plsc.subcore_barrier()
        pltpu.sync_copy(
            acc_s.at[pl.ds(row0, ROWS_PER_TILE)],
            out_hbm.at[c, pl.ds(row0, ROWS_PER_TILE), :])

    return deg_kernel


@functools.cache
def _make_agg(rows, halves):
    DH = 128

    @functools.partial(
        pl.kernel,
        out_type=jax.ShapeDtypeStruct((NC, NP, DH), jnp.float32),
        mesh=_mesh(),
        scratch_types=[
            pltpu.VMEM((CH + 2, CHUNK), jnp.int32),
            pltpu.VMEM((CH, CHUNK), jnp.int32),
            pltpu.VMEM((CHUNK, DH), jnp.float32),
            pltpu.VMEM((CHUNK, DH), jnp.float32),
            pltpu.VMEM_SHARED((NP, DH), jnp.float32),
            pltpu.SemaphoreType.DMA,
            pltpu.SemaphoreType.DMA,
        ],
    )
    def agg_kernel(hs2_hbm, gidx_hbm, didx_hbm, out_hbm,
                   gix_v, dix_v, buf0_v, buf1_v, acc_s, sem0, sem1):
        c = lax.axis_index("c")
        s = lax.axis_index("s")

        @pl.loop(0, CHUNK)
        def _(r):
            for g in range(DH // 16):
                buf0_v[r, pl.ds(g * 16, 16)] = jnp.zeros((16,), jnp.float32)

        row0 = s * ROWS_PER_TILE
        for i in range(ROWS_PER_TILE // CHUNK):
            pltpu.sync_copy(buf0_v, acc_s.at[pl.ds(row0 + i * CHUNK, CHUNK)])
        plsc.subcore_barrier()

        for h in range(halves):
            pltpu.sync_copy(gidx_hbm.at[c, s, h], gix_v)
            pltpu.sync_copy(didx_hbm.at[c, s, h], dix_v)

            pltpu.async_copy(hs2_hbm.at[gix_v.at[0]], buf0_v, sem0)
            pltpu.async_copy(hs2_hbm.at[gix_v.at[1]], buf1_v, sem1)

            @pl.loop(0, CH, step=2)
            def _(jj):
                for b, buf, sem in ((0, buf0_v, sem0), (1, buf1_v, sem1)):
                    j = jj + b
                    pltpu.make_async_copy(
                        hs2_hbm.at[gix_v.at[j]], buf, sem).wait()
                    pltpu.sync_copy(buf, acc_s.at[dix_v.at[j]], add=True)
                    pltpu.async_copy(hs2_hbm.at[gix_v.at[j + 2]], buf, sem)

            pltpu.make_async_copy(
                hs2_hbm.at[gix_v.at[0]], buf0_v, sem0).wait()
            pltpu.make_async_copy(
                hs2_hbm.at[gix_v.at[1]], buf1_v, sem1).wait()

        plsc.subcore_barrier()
        pltpu.sync_copy(
            acc_s.at[pl.ds(row0, ROWS_PER_TILE)],
            out_hbm.at[c, pl.ds(row0, ROWS_PER_TILE), :])

    return agg_kernel


BM = 256
DEG_W = 16


def _dinv_of(deg_ref):
    d = deg_ref[...]
    deg = jnp.sum(d[0] + d[1], axis=1, keepdims=True) + 1.0
    return lax.rsqrt(deg)


def _halves(ref):
    a = ref[...]
    return jnp.concatenate([a[0], a[1]], axis=1)


def _m1_body(x_ref, w_ref, deg_ref, hs_ref):
    hs_ref[...] = _dinv_of(deg_ref) * jnp.dot(
        x_ref[...], w_ref[...],
        preferred_element_type=jnp.float32, precision=lax.Precision.HIGHEST)


def _m2_body(agg_ref, hs_ref, deg_ref, w_ref, b_ref, out_ref):
    dinv = _dinv_of(deg_ref)
    h1 = jnp.maximum(dinv * (_halves(agg_ref) + hs_ref[...]) + b_ref[...], 0.0)
    out_ref[...] = dinv * jnp.dot(
        h1, w_ref[...],
        preferred_element_type=jnp.float32, precision=lax.Precision.HIGHEST)


def _e3_body(agg_ref, hs2_ref, deg_ref, b_ref, out_ref):
    a = agg_ref[...]
    out_ref[...] = (_dinv_of(deg_ref) * (a[0] + a[1] + hs2_ref[...])
                    + b_ref[...])


def _row_spec(d):
    return pl.BlockSpec((BM, d), lambda i: (i, 0))


def _sc_spec(d):
    return pl.BlockSpec((NC, BM, d), lambda i: (0, i, 0))


def _full_spec(r, d):
    return pl.BlockSpec((r, d), lambda i: (0, 0))


def _m1(x, W1, degs):
    return pl.pallas_call(
        _m1_body,
        grid=(NP // BM,),
        in_specs=[_row_spec(D_IN), _full_spec(D_IN, D_H), _sc_spec(DEG_W)],
        out_specs=_row_spec(D_H),
        out_shape=jax.ShapeDtypeStruct((NP, D_H), jnp.float32),
    )(x, W1, degs)


def _m2(agg1, hs, degs, W2, b1):
    return pl.pallas_call(
        _m2_body,
        grid=(NP // BM,),
        in_specs=[_sc_spec(D_H // 2), _row_spec(D_H), _sc_spec(DEG_W),
                  _full_spec(D_H, D_OUT), _full_spec(1, D_H)],
        out_specs=_row_spec(D_OUT),
        out_shape=jax.ShapeDtypeStruct((NP, D_OUT), jnp.float32),
    )(agg1, hs, degs, W2, b1)


def _e3(agg2, hs2, degs, b2):
    return pl.pallas_call(
        _e3_body,
        grid=(NP // BM,),
        in_specs=[_sc_spec(D_OUT), _row_spec(D_OUT), _sc_spec(DEG_W),
                  _full_spec(1, D_OUT)],
        out_specs=_row_spec(D_OUT),
        out_shape=jax.ShapeDtypeStruct((NP, D_OUT), jnp.float32),
    )(agg2, hs2, degs, b2)


def _with_tails(a):
    return jnp.concatenate([a, a[..., :2, :]], axis=-2)


def kernel(x, edge_index, W1, b1, W2, b2):
    src = edge_index[0].astype(jnp.int32)
    dst = edge_index[1].astype(jnp.int32)

    xp = jnp.concatenate(
        [x, jnp.zeros((NP - N, D_IN), jnp.float32)], axis=0)

    pad = EPAD - E
    srcp = jnp.concatenate([src, jnp.zeros((pad,), jnp.int32)])
    dstp = jnp.concatenate([dst, jnp.full((pad,), DUMMY, jnp.int32)])

    g2 = srcp * 2
    a1 = jnp.stack([g2, g2 + 1]).reshape(NC, NS, 2, CH, CHUNK)
    gidx1 = _with_tails(a1)
    d1 = dstp.reshape(NS, 2, CH, CHUNK)
    didx1 = jnp.stack([d1, d1])

    gidx2 = _with_tails(srcp.reshape(NC, NS, 1, CH, CHUNK))
    didx2 = dstp.reshape(NC, NS, 1, CH, CHUNK)

    degs = _make_deg()(didx2.reshape(NC, NS, CH, CHUNK))
    hs = _m1(xp, W1, degs)
    agg1 = _make_agg(2 * NP, 2)(hs.reshape(2 * NP, D_H // 2), gidx1, didx1)
    hs2 = _m2(agg1, hs, degs, W2, b1.reshape(1, D_H))
    agg2 = _make_agg(NP, 1)(hs2, gidx2, didx2)
    out = _e3(agg2, hs2, degs, b2.reshape(1, D_OUT))
    return out[:N]


# speedup vs baseline: 1.0019x; 1.0019x over previous
import functools

import jax
import jax.numpy as jnp
from jax import lax
from jax.experimental import pallas as pl
from jax.experimental.pallas import tpu as pltpu
from jax.experimental.pallas import tpu_sc as plsc

N = 10000
E = 160000
D_IN = 256
D_H = 256
D_OUT = 128

NC = 2
NS = 16
CHUNK = 128
CH = 40

NP = 10240
ROWS_PER_TILE = NP // NS
DUMMY = N

EPAD = NC * NS * CH * CHUNK


@functools.cache
def _mesh():
    return plsc.VectorSubcoreMesh(
        core_axis_name="c", subcore_axis_name="s",
        num_cores=NC, num_subcores=NS)


@functools.cache
def _make_deg():
    DW = 16

    @functools.partial(
        pl.kernel,
        out_type=jax.ShapeDtypeStruct((NC, NP, DW), jnp.float32),
        mesh=_mesh(),
        scratch_types=[
            pltpu.VMEM((CH, CHUNK), jnp.int32),
            pltpu.VMEM((CHUNK, DW), jnp.float32),
            pltpu.VMEM_SHARED((NP, DW), jnp.float32),
        ],
    )
    def deg_kernel(didx_hbm, out_hbm, dix_v, ones_v, acc_s):
        c = lax.axis_index("c")
        s = lax.axis_index("s")
        pltpu.sync_copy(didx_hbm.at[c, s], dix_v)

        @pl.loop(0, CHUNK)
        def _(r):
            ones_v[r, :] = jnp.zeros((DW,), jnp.float32)

        row0 = s * ROWS_PER_TILE
        for i in range(ROWS_PER_TILE // CHUNK):
            pltpu.sync_copy(ones_v, acc_s.at[pl.ds(row0 + i * CHUNK, CHUNK)])

        @pl.loop(0, CHUNK)
        def _(r):
            ones_v[r, :] = jnp.ones((DW,), jnp.float32)

        plsc.subcore_barrier()

        @pl.loop(0, CH)
        def _(j):
            pltpu.sync_copy(ones_v, acc_s.at[dix_v.at[j]], add=True)

        plsc.subcore_barrier()
        pltpu.sync_copy(
            acc_s.at[pl.ds(row0, ROWS_PER_TILE)],
            out_hbm.at[c, pl.ds(row0, ROWS_PER_TILE), :])

    return deg_kernel


@functools.cache
def _make_agg(rows, halves):
    DH = 128

    @functools.partial(
        pl.kernel,
        out_type=jax.ShapeDtypeStruct((NC, NP, DH), jnp.float32),
        mesh=_mesh(),
        scratch_types=[
            pltpu.VMEM((CH + 2, CHUNK), jnp.int32),
            pltpu.VMEM((CH, CHUNK), jnp.int32),
            pltpu.VMEM((CHUNK, DH), jnp.float32),
            pltpu.VMEM((CHUNK, DH), jnp.float32),
            pltpu.VMEM_SHARED((NP, DH), jnp.float32),
            pltpu.SemaphoreType.DMA,
            pltpu.SemaphoreType.DMA,
        ],
    )
    def agg_kernel(hs2_hbm, gidx_hbm, didx_hbm, out_hbm,
                   gix_v, dix_v, buf0_v, buf1_v, acc_s, sem0, sem1):
        c = lax.axis_index("c")
        s = lax.axis_index("s")

        @pl.loop(0, CHUNK)
        def _(r):
            for g in range(DH // 16):
                buf0_v[r, pl.ds(g * 16, 16)] = jnp.zeros((16,), jnp.float32)

        row0 = s * ROWS_PER_TILE
        for i in range(ROWS_PER_TILE // CHUNK):
            pltpu.sync_copy(buf0_v, acc_s.at[pl.ds(row0 + i * CHUNK, CHUNK)])
        plsc.subcore_barrier()

        for h in range(halves):
            pltpu.sync_copy(gidx_hbm.at[c, s, h], gix_v)
            pltpu.sync_copy(didx_hbm.at[c, s, h], dix_v)

            pltpu.async_copy(hs2_hbm.at[gix_v.at[0]], buf0_v, sem0)
            pltpu.async_copy(hs2_hbm.at[gix_v.at[1]], buf1_v, sem1)

            @pl.loop(0, CH, step=2)
            def _(jj):
                for b, buf, sem in ((0, buf0_v, sem0), (1, buf1_v, sem1)):
                    j = jj + b
                    pltpu.make_async_copy(
                        hs2_hbm.at[gix_v.at[j]], buf, sem).wait()
                    pltpu.sync_copy(buf, acc_s.at[dix_v.at[j]], add=True)
                    pltpu.async_copy(hs2_hbm.at[gix_v.at[j + 2]], buf, sem)

            pltpu.make_async_copy(
                hs2_hbm.at[gix_v.at[0]], buf0_v, sem0).wait()
            pltpu.make_async_copy(
                hs2_hbm.at[gix_v.at[1]], buf1_v, sem1).wait()

        plsc.subcore_barrier()
        pltpu.sync_copy(
            acc_s.at[pl.ds(row0, ROWS_PER_TILE)],
            out_hbm.at[c, pl.ds(row0, ROWS_PER_TILE), :])

    return agg_kernel


BM = 256
DEG_W = 16


def _dinv_of(deg_ref):
    d = deg_ref[...]
    deg = jnp.sum(d[0] + d[1], axis=1, keepdims=True) + 1.0
    return lax.rsqrt(deg)


def _halves(ref):
    a = ref[...]
    return jnp.concatenate([a[0], a[1]], axis=1)


def _m1_body(x_ref, w_ref, deg_ref, hs_ref):
    hs_ref[...] = _dinv_of(deg_ref) * jnp.dot(
        x_ref[...], w_ref[...],
        preferred_element_type=jnp.float32, precision=lax.Precision.HIGHEST)


def _m2_body(agg_ref, hs_ref, deg_ref, w_ref, b_ref, out_ref):
    dinv = _dinv_of(deg_ref)
    h1 = jnp.maximum(dinv * (_halves(agg_ref) + hs_ref[...]) + b_ref[...], 0.0)
    out_ref[...] = dinv * jnp.dot(
        h1, w_ref[...],
        preferred_element_type=jnp.float32, precision=lax.Precision.HIGHEST)


def _e3_body(agg_ref, hs2_ref, deg_ref, b_ref, out_ref):
    a = agg_ref[...]
    out_ref[...] = (_dinv_of(deg_ref) * (a[0] + a[1] + hs2_ref[...])
                    + b_ref[...])


def _row_spec(d):
    return pl.BlockSpec((BM, d), lambda i: (i, 0))


def _sc_spec(d):
    return pl.BlockSpec((NC, BM, d), lambda i: (0, i, 0))


def _full_spec(r, d):
    return pl.BlockSpec((r, d), lambda i: (0, 0))


def _m1(x, W1, degs):
    return pl.pallas_call(
        _m1_body,
        grid=(NP // BM,),
        in_specs=[_row_spec(D_IN), _full_spec(D_IN, D_H), _sc_spec(DEG_W)],
        out_specs=_row_spec(D_H),
        out_shape=jax.ShapeDtypeStruct((NP, D_H), jnp.float32),
    )(x, W1, degs)


def _m2(agg1, hs, degs, W2, b1):
    return pl.pallas_call(
        _m2_body,
        grid=(NP // BM,),
        in_specs=[_sc_spec(D_H // 2), _row_spec(D_H), _sc_spec(DEG_W),
                  _full_spec(D_H, D_OUT), _full_spec(1, D_H)],
        out_specs=_row_spec(D_OUT),
        out_shape=jax.ShapeDtypeStruct((NP, D_OUT), jnp.float32),
    )(agg1, hs, degs, W2, b1)


def _e3(agg2, hs2, degs, b2):
    return pl.pallas_call(
        _e3_body,
        grid=(NP // BM,),
        in_specs=[_sc_spec(D_OUT), _row_spec(D_OUT), _sc_spec(DEG_W),
                  _full_spec(1, D_OUT)],
        out_specs=_row_spec(D_OUT),
        out_shape=jax.ShapeDtypeStruct((NP, D_OUT), jnp.float32),
    )(agg2, hs2, degs, b2)


def _with_tails(a):
    return jnp.concatenate([a, a[..., :2, :]], axis=-2)


def kernel(x, edge_index, W1, b1, W2, b2):
    src = edge_index[0].astype(jnp.int32)
    dst = edge_index[1].astype(jnp.int32)

    xp = jnp.concatenate(
        [x, jnp.zeros((NP - N, D_IN), jnp.float32)], axis=0)

    pad = EPAD - E
    srcp = jnp.concatenate([src, jnp.zeros((pad,), jnp.int32)])
    pad_rows = DUMMY + (jnp.arange(pad, dtype=jnp.int32) % (NP - N))
    dstp = jnp.concatenate([dst, pad_rows])

    g2 = srcp * 2
    a1 = jnp.stack([g2, g2 + 1]).reshape(NC, NS, 2, CH, CHUNK)
    gidx1 = _with_tails(a1)
    d1 = dstp.reshape(NS, 2, CH, CHUNK)
    didx1 = jnp.stack([d1, d1])

    gidx2 = _with_tails(srcp.reshape(NC, NS, 1, CH, CHUNK))
    didx2 = dstp.reshape(NC, NS, 1, CH, CHUNK)

    degs = _make_deg()(didx2.reshape(NC, NS, CH, CHUNK))
    hs = _m1(xp, W1, degs)
    agg1 = _make_agg(2 * NP, 2)(hs.reshape(2 * NP, D_H // 2), gidx1, didx1)
    hs2 = _m2(agg1, hs, degs, W2, b1.reshape(1, D_H))
    agg2 = _make_agg(NP, 1)(hs2, gidx2, didx2)
    out = _e3(agg2, hs2, degs, b2.reshape(1, D_OUT))
    return out[:N]


# speedup vs baseline: 1.0885x; 1.0865x over previous
import functools

import jax
import jax.numpy as jnp
from jax import lax
from jax.experimental import pallas as pl
from jax.experimental.pallas import tpu as pltpu
from jax.experimental.pallas import tpu_sc as plsc

N = 10000
E = 160000
D_IN = 256
D_H = 256
D_OUT = 128

NC = 2
NS = 16
CHUNK = 128
CH = 40

NP = 10240
ROWS_PER_TILE = NP // NS
DUMMY = N

EPAD = NC * NS * CH * CHUNK


@functools.cache
def _mesh():
    return plsc.VectorSubcoreMesh(
        core_axis_name="c", subcore_axis_name="s",
        num_cores=NC, num_subcores=NS)


@functools.cache
def _make_deg():
    DW = 128

    @functools.partial(
        pl.kernel,
        out_type=jax.ShapeDtypeStruct((NC, NP, DW), jnp.float32),
        mesh=_mesh(),
        scratch_types=[
            pltpu.VMEM((CH, CHUNK), jnp.int32),
            pltpu.VMEM((CHUNK, DW), jnp.float32),
            pltpu.VMEM_SHARED((NP, DW), jnp.float32),
        ],
    )
    def deg_kernel(didx_hbm, zeros_hbm, ones_hbm, out_hbm, dix_v, ones_v,
                   acc_s):
        c = lax.axis_index("c")
        s = lax.axis_index("s")
        pltpu.sync_copy(didx_hbm.at[c, s], dix_v)
        pltpu.sync_copy(ones_hbm, ones_v)

        row0 = s * ROWS_PER_TILE
        pltpu.sync_copy(zeros_hbm, acc_s.at[pl.ds(row0, ROWS_PER_TILE)])

        plsc.subcore_barrier()

        @pl.loop(0, CH)
        def _(j):
            pltpu.sync_copy(ones_v, acc_s.at[dix_v.at[j]], add=True)

        plsc.subcore_barrier()
        pltpu.sync_copy(
            acc_s.at[pl.ds(row0, ROWS_PER_TILE)],
            out_hbm.at[c, pl.ds(row0, ROWS_PER_TILE), :])

    return deg_kernel


@functools.cache
def _make_agg(rows, halves):
    DH = 128

    @functools.partial(
        pl.kernel,
        out_type=jax.ShapeDtypeStruct((NC, NP, DH), jnp.float32),
        mesh=_mesh(),
        scratch_types=[
            pltpu.VMEM((CH + 2, CHUNK), jnp.int32),
            pltpu.VMEM((CH, CHUNK), jnp.int32),
            pltpu.VMEM((CHUNK, DH), jnp.float32),
            pltpu.VMEM((CHUNK, DH), jnp.float32),
            pltpu.VMEM_SHARED((NP, DH), jnp.float32),
            pltpu.SemaphoreType.DMA,
            pltpu.SemaphoreType.DMA,
        ],
    )
    def agg_kernel(hs2_hbm, gidx_hbm, didx_hbm, zeros_hbm, out_hbm,
                   gix_v, dix_v, buf0_v, buf1_v, acc_s, sem0, sem1):
        c = lax.axis_index("c")
        s = lax.axis_index("s")

        row0 = s * ROWS_PER_TILE
        pltpu.sync_copy(zeros_hbm, acc_s.at[pl.ds(row0, ROWS_PER_TILE)])
        plsc.subcore_barrier()

        for h in range(halves):
            pltpu.sync_copy(gidx_hbm.at[c, s, h], gix_v)
            pltpu.sync_copy(didx_hbm.at[c, s, h], dix_v)

            pltpu.async_copy(hs2_hbm.at[gix_v.at[0]], buf0_v, sem0)
            pltpu.async_copy(hs2_hbm.at[gix_v.at[1]], buf1_v, sem1)

            @pl.loop(0, CH, step=2)
            def _(jj):
                for b, buf, sem in ((0, buf0_v, sem0), (1, buf1_v, sem1)):
                    j = jj + b
                    pltpu.make_async_copy(
                        hs2_hbm.at[gix_v.at[j]], buf, sem).wait()
                    pltpu.sync_copy(buf, acc_s.at[dix_v.at[j]], add=True)
                    pltpu.async_copy(hs2_hbm.at[gix_v.at[j + 2]], buf, sem)

            pltpu.make_async_copy(
                hs2_hbm.at[gix_v.at[0]], buf0_v, sem0).wait()
            pltpu.make_async_copy(
                hs2_hbm.at[gix_v.at[1]], buf1_v, sem1).wait()

        plsc.subcore_barrier()
        pltpu.sync_copy(
            acc_s.at[pl.ds(row0, ROWS_PER_TILE)],
            out_hbm.at[c, pl.ds(row0, ROWS_PER_TILE), :])

    return agg_kernel


BM = 256
DEG_W = 128


def _dinv_of(deg_ref):
    d = deg_ref[...]
    deg = d[0, :, :1] + d[1, :, :1] + 1.0
    return lax.rsqrt(deg)


def _halves(ref):
    a = ref[...]
    return jnp.concatenate([a[0], a[1]], axis=1)


def _m1_body(x_ref, w_ref, deg_ref, hs_ref):
    hs_ref[...] = _dinv_of(deg_ref) * jnp.dot(
        x_ref[...], w_ref[...],
        preferred_element_type=jnp.float32, precision=lax.Precision.HIGHEST)


def _m2_body(agg_ref, hs_ref, deg_ref, w_ref, b_ref, out_ref):
    dinv = _dinv_of(deg_ref)
    h1 = jnp.maximum(dinv * (_halves(agg_ref) + hs_ref[...]) + b_ref[...], 0.0)
    out_ref[...] = dinv * jnp.dot(
        h1, w_ref[...],
        preferred_element_type=jnp.float32, precision=lax.Precision.HIGHEST)


def _e3_body(agg_ref, hs2_ref, deg_ref, b_ref, out_ref):
    a = agg_ref[...]
    out_ref[...] = (_dinv_of(deg_ref) * (a[0] + a[1] + hs2_ref[...])
                    + b_ref[...])


def _row_spec(d):
    return pl.BlockSpec((BM, d), lambda i: (i, 0))


def _sc_spec(d):
    return pl.BlockSpec((NC, BM, d), lambda i: (0, i, 0))


def _full_spec(r, d):
    return pl.BlockSpec((r, d), lambda i: (0, 0))


def _m1(x, W1, degs):
    return pl.pallas_call(
        _m1_body,
        grid=(NP // BM,),
        in_specs=[_row_spec(D_IN), _full_spec(D_IN, D_H), _sc_spec(DEG_W)],
        out_specs=_row_spec(D_H),
        out_shape=jax.ShapeDtypeStruct((NP, D_H), jnp.float32),
    )(x, W1, degs)


def _m2(agg1, hs, degs, W2, b1):
    return pl.pallas_call(
        _m2_body,
        grid=(NP // BM,),
        in_specs=[_sc_spec(D_H // 2), _row_spec(D_H), _sc_spec(DEG_W),
                  _full_spec(D_H, D_OUT), _full_spec(1, D_H)],
        out_specs=_row_spec(D_OUT),
        out_shape=jax.ShapeDtypeStruct((NP, D_OUT), jnp.float32),
    )(agg1, hs, degs, W2, b1)


def _e3(agg2, hs2, degs, b2):
    return pl.pallas_call(
        _e3_body,
        grid=(NP // BM,),
        in_specs=[_sc_spec(D_OUT), _row_spec(D_OUT), _sc_spec(DEG_W),
                  _full_spec(1, D_OUT)],
        out_specs=_row_spec(D_OUT),
        out_shape=jax.ShapeDtypeStruct((NP, D_OUT), jnp.float32),
    )(agg2, hs2, degs, b2)


def _with_tails(a):
    return jnp.concatenate([a, a[..., :2, :]], axis=-2)


def kernel(x, edge_index, W1, b1, W2, b2):
    src = edge_index[0].astype(jnp.int32)
    dst = edge_index[1].astype(jnp.int32)

    xp = jnp.concatenate(
        [x, jnp.zeros((NP - N, D_IN), jnp.float32)], axis=0)

    pad = EPAD - E
    srcp = jnp.concatenate([src, jnp.zeros((pad,), jnp.int32)])
    pad_rows = DUMMY + (jnp.arange(pad, dtype=jnp.int32) % (NP - N))
    dstp = jnp.concatenate([dst, pad_rows])

    g2 = srcp * 2
    a1 = jnp.stack([g2, g2 + 1]).reshape(NC, NS, 2, CH, CHUNK)
    gidx1 = _with_tails(a1)
    d1 = dstp.reshape(NS, 2, CH, CHUNK)
    didx1 = jnp.stack([d1, d1])

    gidx2 = _with_tails(srcp.reshape(NC, NS, 1, CH, CHUNK))
    didx2 = dstp.reshape(NC, NS, 1, CH, CHUNK)

    zeros_w = jnp.zeros((ROWS_PER_TILE, 128), jnp.float32)
    ones_d = jnp.ones((CHUNK, 128), jnp.float32)

    degs = _make_deg()(didx2.reshape(NC, NS, CH, CHUNK), zeros_w, ones_d)
    hs = _m1(xp, W1, degs)
    agg1 = _make_agg(2 * NP, 2)(
        hs.reshape(2 * NP, D_H // 2), gidx1, didx1, zeros_w)
    hs2 = _m2(agg1, hs, degs, W2, b1.reshape(1, D_H))
    agg2 = _make_agg(NP, 1)(hs2, gidx2, didx2, zeros_w)
    out = _e3(agg2, hs2, degs, b2.reshape(1, D_OUT))
    return out[:N]


# speedup vs baseline: 1.8946x; 1.7405x over previous
import functools

import jax
import jax.numpy as jnp
from jax import lax
from jax.experimental import pallas as pl
from jax.experimental.pallas import tpu as pltpu
from jax.experimental.pallas import tpu_sc as plsc

N = 10000
E = 160000
D_IN = 256
D_H = 256
D_OUT = 128

NC = 2
NS = 16
CHUNK = 128
CH = 40

NP = 10240
ROWS_PER_TILE = NP // NS
DUMMY = N

EPAD = NC * NS * CH * CHUNK


@functools.cache
def _mesh():
    return plsc.VectorSubcoreMesh(
        core_axis_name="c", subcore_axis_name="s",
        num_cores=NC, num_subcores=NS)


@functools.cache
def _make_deg():
    DW = 128

    @functools.partial(
        pl.kernel,
        out_type=jax.ShapeDtypeStruct((NC, NP, DW), jnp.float32),
        mesh=_mesh(),
        scratch_types=[
            pltpu.VMEM((CH, CHUNK), jnp.int32),
            pltpu.VMEM((CHUNK, DW), jnp.float32),
            pltpu.VMEM_SHARED((NP, DW), jnp.float32),
        ],
    )
    def deg_kernel(didx_hbm, zeros_hbm, ones_hbm, out_hbm, dix_v, ones_v,
                   acc_s):
        c = lax.axis_index("c")
        s = lax.axis_index("s")
        pltpu.sync_copy(didx_hbm.at[c, s], dix_v)
        pltpu.sync_copy(ones_hbm, ones_v)

        row0 = s * ROWS_PER_TILE
        pltpu.sync_copy(zeros_hbm, acc_s.at[pl.ds(row0, ROWS_PER_TILE)])

        plsc.subcore_barrier()

        @pl.loop(0, CH)
        def _(j):
            pltpu.sync_copy(ones_v, acc_s.at[dix_v.at[j]], add=True)

        plsc.subcore_barrier()
        pltpu.sync_copy(
            acc_s.at[pl.ds(row0, ROWS_PER_TILE)],
            out_hbm.at[c, pl.ds(row0, ROWS_PER_TILE), :])

    return deg_kernel


@functools.cache
def _make_agg(rows, halves):
    DH = 128

    @functools.partial(
        pl.kernel,
        out_type=jax.ShapeDtypeStruct((NC, NP, DH), jnp.float32),
        mesh=_mesh(),
        scratch_types=[
            pltpu.VMEM((CH + 2, CHUNK), jnp.int32),
            pltpu.VMEM((CH, CHUNK), jnp.int32),
            pltpu.VMEM((CHUNK, DH), jnp.float32),
            pltpu.VMEM((CHUNK, DH), jnp.float32),
            pltpu.VMEM_SHARED((NP, DH), jnp.float32),
            pltpu.SemaphoreType.DMA,
            pltpu.SemaphoreType.DMA,
        ],
    )
    def agg_kernel(hs2_hbm, gidx_hbm, didx_hbm, zeros_hbm, out_hbm,
                   gix_v, dix_v, buf0_v, buf1_v, acc_s, sem0, sem1):
        c = lax.axis_index("c")
        s = lax.axis_index("s")

        row0 = s * ROWS_PER_TILE
        pltpu.sync_copy(zeros_hbm, acc_s.at[pl.ds(row0, ROWS_PER_TILE)])
        plsc.subcore_barrier()

        for h in range(halves):
            pltpu.sync_copy(gidx_hbm.at[c, s, h], gix_v)
            pltpu.sync_copy(didx_hbm.at[c, s, h], dix_v)

            pltpu.async_copy(hs2_hbm.at[gix_v.at[0]], buf0_v, sem0)
            pltpu.async_copy(hs2_hbm.at[gix_v.at[1]], buf1_v, sem1)

            @pl.loop(0, CH, step=2)
            def _(jj):
                for b, buf, sem in ((0, buf0_v, sem0), (1, buf1_v, sem1)):
                    j = jj + b
                    pltpu.make_async_copy(
                        hs2_hbm.at[gix_v.at[j]], buf, sem).wait()
                    pltpu.sync_copy(buf, acc_s.at[dix_v.at[j]], add=True)
                    pltpu.async_copy(hs2_hbm.at[gix_v.at[j + 2]], buf, sem)

            pltpu.make_async_copy(
                hs2_hbm.at[gix_v.at[0]], buf0_v, sem0).wait()
            pltpu.make_async_copy(
                hs2_hbm.at[gix_v.at[1]], buf1_v, sem1).wait()

        plsc.subcore_barrier()
        pltpu.sync_copy(
            acc_s.at[pl.ds(row0, ROWS_PER_TILE)],
            out_hbm.at[c, pl.ds(row0, ROWS_PER_TILE), :])

    return agg_kernel


BM = 256
DEG_W = 128


def _dinv_of(deg_ref):
    d = deg_ref[...]
    deg = d[0, :, :1] + d[1, :, :1] + 1.0
    return lax.rsqrt(deg)


def _halves(ref):
    a = ref[...]
    return jnp.concatenate([a[0], a[1]], axis=1)


def _m1_body(x_ref, w_ref, deg_ref, hs_ref):
    hs_ref[...] = _dinv_of(deg_ref) * jnp.dot(
        x_ref[...], w_ref[...],
        preferred_element_type=jnp.float32, precision=lax.Precision.HIGHEST)


def _m2_body(agg_ref, hs_ref, deg_ref, w_ref, b_ref, out_ref):
    dinv = _dinv_of(deg_ref)
    h1 = jnp.maximum(dinv * (_halves(agg_ref) + hs_ref[...]) + b_ref[...], 0.0)
    out_ref[...] = dinv * jnp.dot(
        h1, w_ref[...],
        preferred_element_type=jnp.float32, precision=lax.Precision.HIGHEST)


def _e3_body(agg_ref, hs2_ref, deg_ref, b_ref, out_ref):
    a = agg_ref[...]
    out_ref[...] = (_dinv_of(deg_ref) * (a[0] + a[1] + hs2_ref[...])
                    + b_ref[...])


def _row_spec(d):
    return pl.BlockSpec((BM, d), lambda i: (i, 0))


def _sc_spec(d):
    return pl.BlockSpec((NC, BM, d), lambda i: (0, i, 0))


def _full_spec(r, d):
    return pl.BlockSpec((r, d), lambda i: (0, 0))


def _m1(x, W1, degs):
    return pl.pallas_call(
        _m1_body,
        grid=(NP // BM,),
        in_specs=[_row_spec(D_IN), _full_spec(D_IN, D_H), _sc_spec(DEG_W)],
        out_specs=_row_spec(D_H),
        out_shape=jax.ShapeDtypeStruct((NP, D_H), jnp.float32),
    )(x, W1, degs)


def _m2(agg1, hs, degs, W2, b1):
    return pl.pallas_call(
        _m2_body,
        grid=(NP // BM,),
        in_specs=[_sc_spec(D_H // 2), _row_spec(D_H), _sc_spec(DEG_W),
                  _full_spec(D_H, D_OUT), _full_spec(1, D_H)],
        out_specs=_row_spec(D_OUT),
        out_shape=jax.ShapeDtypeStruct((NP, D_OUT), jnp.float32),
    )(agg1, hs, degs, W2, b1)


def _e3(agg2, hs2, degs, b2):
    return pl.pallas_call(
        _e3_body,
        grid=(NP // BM,),
        in_specs=[_sc_spec(D_OUT), _row_spec(D_OUT), _sc_spec(DEG_W),
                  _full_spec(1, D_OUT)],
        out_specs=_row_spec(D_OUT),
        out_shape=jax.ShapeDtypeStruct((NP, D_OUT), jnp.float32),
    )(agg2, hs2, degs, b2)


def _with_tails(a):
    return jnp.concatenate([a, a[..., :2, :]], axis=-2)


def kernel(x, edge_index, W1, b1, W2, b2):
    src = edge_index[0].astype(jnp.int32)
    dst = edge_index[1].astype(jnp.int32)

    xp = jnp.concatenate(
        [x, jnp.zeros((NP - N, D_IN), jnp.float32)], axis=0)

    pad = EPAD - E
    pad_i = jnp.arange(pad, dtype=jnp.int32)
    srcp = jnp.concatenate([src, pad_i % N])
    dstp = jnp.concatenate([dst, DUMMY + pad_i % (NP - N)])

    g2 = srcp * 2
    a1 = jnp.stack([g2, g2 + 1]).reshape(NC, NS, 2, CH, CHUNK)
    gidx1 = _with_tails(a1)
    d1 = dstp.reshape(NS, 2, CH, CHUNK)
    didx1 = jnp.stack([d1, d1])

    gidx2 = _with_tails(srcp.reshape(NC, NS, 1, CH, CHUNK))
    didx2 = dstp.reshape(NC, NS, 1, CH, CHUNK)

    zeros_w = jnp.zeros((ROWS_PER_TILE, 128), jnp.float32)
    ones_d = jnp.ones((CHUNK, 128), jnp.float32)

    degs = _make_deg()(didx2.reshape(NC, NS, CH, CHUNK), zeros_w, ones_d)
    hs = _m1(xp, W1, degs)
    agg1 = _make_agg(2 * NP, 2)(
        hs.reshape(2 * NP, D_H // 2), gidx1, didx1, zeros_w)
    hs2 = _m2(agg1, hs, degs, W2, b1.reshape(1, D_H))
    agg2 = _make_agg(NP, 1)(hs2, gidx2, didx2, zeros_w)
    out = _e3(agg2, hs2, degs, b2.reshape(1, D_OUT))
    return out[:N]


# speedup vs baseline: 2.1040x; 1.1106x over previous
import functools

import jax
import jax.numpy as jnp
from jax import lax
from jax.experimental import pallas as pl
from jax.experimental.pallas import tpu as pltpu
from jax.experimental.pallas import tpu_sc as plsc

N = 10000
E = 160000
D_IN = 256
D_H = 256
D_OUT = 128

NC = 2
NS = 16
CHUNK = 128
CH = 40

NP = 10240
ROWS_PER_TILE = NP // NS
DUMMY = N

EPAD = NC * NS * CH * CHUNK


@functools.cache
def _mesh():
    return plsc.VectorSubcoreMesh(
        core_axis_name="c", subcore_axis_name="s",
        num_cores=NC, num_subcores=NS)


@functools.cache
def _make_deg():
    DW = 128

    @functools.partial(
        pl.kernel,
        out_type=jax.ShapeDtypeStruct((NC, NP, DW), jnp.float32),
        mesh=_mesh(),
        scratch_types=[
            pltpu.VMEM((CH, CHUNK), jnp.int32),
            pltpu.VMEM((CHUNK, DW), jnp.float32),
            pltpu.VMEM_SHARED((NP, DW), jnp.float32),
        ],
    )
    def deg_kernel(didx_hbm, zeros_hbm, ones_hbm, out_hbm, dix_v, ones_v,
                   acc_s):
        c = lax.axis_index("c")
        s = lax.axis_index("s")
        pltpu.sync_copy(didx_hbm.at[c, s], dix_v)
        pltpu.sync_copy(ones_hbm, ones_v)

        row0 = s * ROWS_PER_TILE
        pltpu.sync_copy(zeros_hbm, acc_s.at[pl.ds(row0, ROWS_PER_TILE)])

        plsc.subcore_barrier()

        @pl.loop(0, CH)
        def _(j):
            pltpu.sync_copy(ones_v, acc_s.at[dix_v.at[j]], add=True)

        plsc.subcore_barrier()
        pltpu.sync_copy(
            acc_s.at[pl.ds(row0, ROWS_PER_TILE)],
            out_hbm.at[c, pl.ds(row0, ROWS_PER_TILE), :])

    return deg_kernel


@functools.cache
def _make_agg(rows, halves):
    DH = 128

    @functools.partial(
        pl.kernel,
        out_type=jax.ShapeDtypeStruct((NC, NP, DH), jnp.float32),
        mesh=_mesh(),
        scratch_types=[
            pltpu.VMEM((CH + 2, CHUNK), jnp.int32),
            pltpu.VMEM((CH, CHUNK), jnp.int32),
            pltpu.VMEM((CHUNK, DH), jnp.float32),
            pltpu.VMEM((CHUNK, DH), jnp.float32),
            pltpu.VMEM_SHARED((NP, DH), jnp.float32),
            pltpu.SemaphoreType.DMA,
            pltpu.SemaphoreType.DMA,
        ],
    )
    def agg_kernel(hs2_hbm, gidx_hbm, didx_hbm, zeros_hbm, out_hbm,
                   gix_v, dix_v, buf0_v, buf1_v, acc_s, sem0, sem1):
        c = lax.axis_index("c")
        s = lax.axis_index("s")

        row0 = s * ROWS_PER_TILE
        pltpu.sync_copy(zeros_hbm, acc_s.at[pl.ds(row0, ROWS_PER_TILE)])
        plsc.subcore_barrier()

        for h in range(halves):
            pltpu.sync_copy(gidx_hbm.at[c, s, h], gix_v)
            pltpu.sync_copy(didx_hbm.at[c, s, h], dix_v)

            pltpu.async_copy(hs2_hbm.at[gix_v.at[0]], buf0_v, sem0)
            pltpu.async_copy(hs2_hbm.at[gix_v.at[1]], buf1_v, sem1)

            @pl.loop(0, CH, step=2)
            def _(jj):
                for b, buf, sem in ((0, buf0_v, sem0), (1, buf1_v, sem1)):
                    j = jj + b
                    pltpu.make_async_copy(
                        hs2_hbm.at[gix_v.at[j]], buf, sem).wait()
                    pltpu.sync_copy(buf, acc_s.at[dix_v.at[j]], add=True)
                    pltpu.async_copy(hs2_hbm.at[gix_v.at[j + 2]], buf, sem)

            pltpu.make_async_copy(
                hs2_hbm.at[gix_v.at[0]], buf0_v, sem0).wait()
            pltpu.make_async_copy(
                hs2_hbm.at[gix_v.at[1]], buf1_v, sem1).wait()

        plsc.subcore_barrier()
        pltpu.sync_copy(
            acc_s.at[pl.ds(row0, ROWS_PER_TILE)],
            out_hbm.at[c, pl.ds(row0, ROWS_PER_TILE), :])

    return agg_kernel


BM = 512
DEG_W = 128


def _dinv_of(deg_ref):
    d = deg_ref[...]
    deg = d[0, :, :1] + d[1, :, :1] + 1.0
    return lax.rsqrt(deg)


def _halves(ref):
    a = ref[...]
    return jnp.concatenate([a[0], a[1]], axis=1)


def _m1a_body(x_ref, w_ref, u_ref):
    u_ref[...] = jnp.dot(
        x_ref[...], w_ref[...],
        preferred_element_type=jnp.float32, precision=lax.Precision.HIGHEST)


def _m1b_body(u_ref, deg_ref, hs_ref):
    hs_ref[...] = _dinv_of(deg_ref) * u_ref[...]


def _m2_body(agg_ref, hs_ref, deg_ref, w_ref, b_ref, out_ref):
    dinv = _dinv_of(deg_ref)
    h1 = jnp.maximum(dinv * (_halves(agg_ref) + hs_ref[...]) + b_ref[...], 0.0)
    out_ref[...] = dinv * jnp.dot(
        h1, w_ref[...],
        preferred_element_type=jnp.float32, precision=lax.Precision.HIGHEST)


def _e3_body(agg_ref, hs2_ref, deg_ref, b_ref, out_ref):
    a = agg_ref[...]
    out_ref[...] = (_dinv_of(deg_ref) * (a[0] + a[1] + hs2_ref[...])
                    + b_ref[...])


def _row_spec(d):
    return pl.BlockSpec((BM, d), lambda i: (i, 0))


def _sc_spec(d):
    return pl.BlockSpec((NC, BM, d), lambda i: (0, i, 0))


def _full_spec(r, d):
    return pl.BlockSpec((r, d), lambda i: (0, 0))


def _m1a(x, W1):
    return pl.pallas_call(
        _m1a_body,
        grid=(NP // BM,),
        in_specs=[_row_spec(D_IN), _full_spec(D_IN, D_H)],
        out_specs=_row_spec(D_H),
        out_shape=jax.ShapeDtypeStruct((NP, D_H), jnp.float32),
    )(x, W1)


def _m1b(u, degs):
    return pl.pallas_call(
        _m1b_body,
        grid=(NP // BM,),
        in_specs=[_row_spec(D_H), _sc_spec(DEG_W)],
        out_specs=_row_spec(D_H),
        out_shape=jax.ShapeDtypeStruct((NP, D_H), jnp.float32),
    )(u, degs)


def _m2(agg1, hs, degs, W2, b1):
    return pl.pallas_call(
        _m2_body,
        grid=(NP // BM,),
        in_specs=[_sc_spec(D_H // 2), _row_spec(D_H), _sc_spec(DEG_W),
                  _full_spec(D_H, D_OUT), _full_spec(1, D_H)],
        out_specs=_row_spec(D_OUT),
        out_shape=jax.ShapeDtypeStruct((NP, D_OUT), jnp.float32),
    )(agg1, hs, degs, W2, b1)


def _e3(agg2, hs2, degs, b2):
    return pl.pallas_call(
        _e3_body,
        grid=(NP // BM,),
        in_specs=[_sc_spec(D_OUT), _row_spec(D_OUT), _sc_spec(DEG_W),
                  _full_spec(1, D_OUT)],
        out_specs=_row_spec(D_OUT),
        out_shape=jax.ShapeDtypeStruct((NP, D_OUT), jnp.float32),
    )(agg2, hs2, degs, b2)


def _with_tails(a):
    return jnp.concatenate([a, a[..., :2, :]], axis=-2)


def kernel(x, edge_index, W1, b1, W2, b2):
    src = edge_index[0].astype(jnp.int32)
    dst = edge_index[1].astype(jnp.int32)

    xp = jnp.concatenate(
        [x, jnp.zeros((NP - N, D_IN), jnp.float32)], axis=0)

    pad = EPAD - E
    pad_i = jnp.arange(pad, dtype=jnp.int32)
    srcp = jnp.concatenate([src, pad_i % N])
    dstp = jnp.concatenate([dst, DUMMY + pad_i % (NP - N)])

    g2 = srcp * 2
    a1 = jnp.stack([g2, g2 + 1]).reshape(NC, NS, 2, CH, CHUNK)
    gidx1 = _with_tails(a1)
    d1 = dstp.reshape(NS, 2, CH, CHUNK)
    didx1 = jnp.stack([d1, d1])

    gidx2 = _with_tails(srcp.reshape(NC, NS, 1, CH, CHUNK))
    didx2 = dstp.reshape(NC, NS, 1, CH, CHUNK)

    zeros_w = jnp.zeros((ROWS_PER_TILE, 128), jnp.float32)
    ones_d = jnp.ones((CHUNK, 128), jnp.float32)

    degs = _make_deg()(didx2.reshape(NC, NS, CH, CHUNK), zeros_w, ones_d)
    u = _m1a(xp, W1)
    hs = _m1b(u, degs)
    agg1 = _make_agg(2 * NP, 2)(
        hs.reshape(2 * NP, D_H // 2), gidx1, didx1, zeros_w)
    hs2 = _m2(agg1, hs, degs, W2, b1.reshape(1, D_H))
    agg2 = _make_agg(NP, 1)(hs2, gidx2, didx2, zeros_w)
    out = _e3(agg2, hs2, degs, b2.reshape(1, D_OUT))
    return out[:N]
